# baseline (speedup 1.0000x reference)
import functools

import jax
import jax.numpy as jnp
from jax.experimental import pallas as pl
from jax.experimental.pallas import tpu as pltpu

_H = 8
_H2 = 64
_CNN_K, _CNN_S = 3, 2

_CH = 1024
_U = 1


def _gcn_kernel(w_ref, x_ref, ws_ref, bs_ref,
                wg0_ref, bg0_ref, m00w_ref, m00b_ref, m01w_ref, m01b_ref,
                wg1_ref, bg1_ref, m10w_ref, m10b_ref, m11w_ref, m11b_ref,
                o_ref, ae_ref, ao_ref, k0_ref, k1_ref, sem_ref, *, n):
    f32 = jnp.float32
    segs = n // 128
    n_edges = w_ref.shape[1] * w_ref.shape[2]
    nch = n_edges // _CH

    ae_ref[...] = jnp.zeros((segs * n, 128), f32)
    ao_ref[...] = jnp.zeros((segs * n, 128), f32)

    iota_sl = ((jax.lax.broadcasted_iota(jnp.int32, (8, 128), 0) << 7)
               | jax.lax.broadcasted_iota(jnp.int32, (8, 128), 1))

    def edge_copy(c, k_ref, s):
        return pltpu.make_async_copy(
            w_ref.at[0, pl.ds(c * 8, 8), :], k_ref, sem_ref.at[s])

    def do_chunk(k_ref):
        for g in range(0, _CH, _U):
            ab = ae_ref if (g // _U) % 2 == 0 else ao_ref
            wv = [k_ref[(g + i) // 128, (g + i) % 128] for i in range(_U)]
            bases = [pl.multiple_of(w >> 10, 8) for w in wv]
            keys = [w & 1023 for w in wv]
            curs = [ab[pl.ds(bases[i], 8), :] for i in range(_U)]
            news = []
            for i in range(_U):
                val = curs[i]
                for j in range(i):
                    val = jnp.where(bases[j] == bases[i], news[j], val)
                news.append(jnp.where(iota_sl == keys[i], 1.0, val))
            for i in range(_U):
                ab[pl.ds(bases[i], 8), :] = news[i]

    edge_copy(0, k0_ref, 0).start()
    if nch > 1:
        edge_copy(1, k1_ref, 1).start()

    def ebody(c2, carry):
        c = c2 * 2
        edge_copy(c, k0_ref, 0).wait()
        do_chunk(k0_ref)

        @pl.when(c + 2 < nch)
        def _():
            edge_copy(c + 2, k0_ref, 0).start()

        edge_copy(c + 1, k1_ref, 1).wait()
        do_chunk(k1_ref)

        @pl.when(c + 3 < nch)
        def _():
            edge_copy(c + 3, k1_ref, 1).start()

        return carry

    jax.lax.fori_loop(0, nch // 2, ebody, 0)
    if nch % 2 == 1:
        edge_copy(nch - 1, k0_ref, 0).wait()
        do_chunk(k0_ref)

    ae_ref[...] = jnp.maximum(ae_ref[...], ao_ref[...])

    x = x_ref[0]
    h0 = jnp.maximum(
        jnp.dot(x, ws_ref[...], preferred_element_type=f32) + bs_ref[...], 0.0)

    deg = None
    for s in range(segs):
        blk = ae_ref[s * n:(s + 1) * n, :]
        r = jnp.sum(blk, axis=1, keepdims=True)
        deg = r if deg is None else deg + r
    dinv = jax.lax.rsqrt(deg + 1.0)

    def aggmul(v):
        acc = None
        for s in range(segs):
            blk = ae_ref[s * n:(s + 1) * n, :]
            p = jnp.dot(blk, v[s * 128:(s + 1) * 128, :],
                        preferred_element_type=f32)
            acc = p if acc is None else acc + p
        return acc

    v0 = jnp.dot(h0, wg0_ref[...], preferred_element_type=f32) * dinv
    agg = (aggmul(v0) + v0) * dinv + bg0_ref[...]
    m = jnp.maximum(
        jnp.dot(agg, m00w_ref[...], preferred_element_type=f32) + m00b_ref[...], 0.0)
    h1 = jnp.dot(m, m01w_ref[...], preferred_element_type=f32) + m01b_ref[...]

    v1 = jnp.dot(h1, wg1_ref[...], preferred_element_type=f32) * dinv
    agg1 = (aggmul(v1) + v1) * dinv + bg1_ref[...]
    m1 = jnp.maximum(
        jnp.dot(agg1, m10w_ref[...], preferred_element_type=f32) + m10b_ref[...], 0.0)
    h2 = jnp.dot(m1, m11w_ref[...], preferred_element_type=f32) + m11b_ref[...]

    inv_n = 1.0 / n
    p1 = jnp.sum(h1, axis=0, keepdims=True) * inv_n
    p2 = jnp.sum(h2, axis=0, keepdims=True) * inv_n
    o_ref[0] = jnp.concatenate([p1, p2], axis=1)


def _gcn_stack(w_blocks, x_blocks, ws, bs, wg0, bg0, m00w, m00b, m01w, m01b,
               wg1, bg1, m10w, m10b, m11w, m11b):
    b, n, f = x_blocks.shape
    segs = n // 128
    ew = w_blocks.shape[1]
    wspec = lambda shape: pl.BlockSpec(shape, lambda i: (0, 0))
    return pl.pallas_call(
        functools.partial(_gcn_kernel, n=n),
        out_shape=jax.ShapeDtypeStruct((b, 1, 2 * _H2), jnp.float32),
        grid=(b,),
        in_specs=[
            pl.BlockSpec((1, ew, 128), lambda i: (i, 0, 0)),
            pl.BlockSpec((1, n, f), lambda i: (i, 0, 0)),
            wspec((f, _H)), wspec((1, _H)),
            wspec((_H, _H)), wspec((1, _H)),
            wspec((_H, _H)), wspec((1, _H)), wspec((_H, _H2)), wspec((1, _H2)),
            wspec((_H2, _H)), wspec((1, _H)),
            wspec((_H, _H)), wspec((1, _H)), wspec((_H, _H2)), wspec((1, _H2)),
        ],
        out_specs=pl.BlockSpec((1, 1, 2 * _H2), lambda i: (i, 0, 0)),
        scratch_shapes=[
            pltpu.VMEM((segs * n, 128), jnp.float32),
            pltpu.VMEM((segs * n, 128), jnp.float32),
            pltpu.SMEM((8, 128), jnp.int32),
            pltpu.SMEM((8, 128), jnp.int32),
            pltpu.SemaphoreType.DMA((2,)),
        ],
        compiler_params=pltpu.CompilerParams(
            dimension_semantics=("parallel",),
            vmem_limit_bytes=60 * 1024 * 1024),
        cost_estimate=pl.CostEstimate(
            flops=2 * b * (2 * n * n * _H + n * (f + 4 * _H) * _H
                           + 2 * n * _H * _H2),
            transcendentals=0,
            bytes_accessed=4 * b * (ew * 128 + n * f + 2 * _H2)),
    )(w_blocks, x_blocks, ws, bs.reshape(1, _H),
      wg0, bg0.reshape(1, _H), m00w, m00b.reshape(1, _H),
      m01w, m01b.reshape(1, _H2),
      wg1, bg1.reshape(1, _H), m10w, m10b.reshape(1, _H),
      m11w, m11b.reshape(1, _H2))


def _conv_kernel(c_ref, w_ref, b_ref, o_ref):
    acc = jnp.dot(c_ref[...], w_ref[...], preferred_element_type=jnp.float32)
    o_ref[...] = jnp.maximum(acc + b_ref[...], 0.0)


def _conv_gemm(col, w, bias):
    r, k = col.shape
    return pl.pallas_call(
        _conv_kernel,
        out_shape=jax.ShapeDtypeStruct((r, _H), jnp.float32),
        grid=(1,),
        in_specs=[
            pl.BlockSpec((r, k), lambda i: (0, 0)),
            pl.BlockSpec((k, _H), lambda i: (0, 0)),
            pl.BlockSpec((1, _H), lambda i: (0, 0)),
        ],
        out_specs=pl.BlockSpec((r, _H), lambda i: (0, 0)),
        compiler_params=pltpu.CompilerParams(
            dimension_semantics=("arbitrary",)),
    )(col, w, bias.reshape(1, _H))


def _tail_kernel(pool_ref, pif_ref, kg_ref, kp_ref, ko_ref,
                 ss_ref, sd_ref, bs_ref, bd_ref,
                 w1_ref, b1_ref, w2_ref, b2_ref,
                 o_ref, og_ref, ot_ref):
    f32 = jnp.float32
    gcn_cat = jnp.dot(pool_ref[...], kg_ref[...], preferred_element_type=f32)
    pi_cat = jnp.dot(pif_ref[...], kp_ref[...], preferred_element_type=f32)
    z_gcn = gcn_cat[:, _H * _H2:_H * _H2 + _H2]
    z_pi = pi_cat[:, _H * _H2:_H * _H2 + _H2]
    z_dual = (
        jnp.dot(gcn_cat[:, :_H * _H2], ko_ref[:_H * _H2, :],
                preferred_element_type=f32)
        + jnp.dot(pi_cat[:, :_H * _H2], ko_ref[_H * _H2:, :],
                  preferred_element_type=f32))

    def head(z, b_att, s):
        att = jnp.maximum(z + b_att, 0.0)
        pooled = jnp.dot(att, s, preferred_element_type=f32)
        hdn = jnp.maximum(
            jnp.dot(pooled, w1_ref[...], preferred_element_type=f32)
            + b1_ref[...], 0.0)
        return jnp.dot(hdn, w2_ref[...], preferred_element_type=f32) + b2_ref[...]

    o_ref[...] = head(z_dual, bd_ref[...], sd_ref[...])
    og_ref[...] = head(z_gcn, bs_ref[...], ss_ref[...])
    ot_ref[...] = head(z_pi, bs_ref[...], ss_ref[...])


def _tail(pooled, pi_flat, k_gcn_cat, k_pi_r, k_out,
          s_single, s_dual, b_single, b_dual, w1, b1, w2, b2):
    b = pooled.shape[0]
    pi_in = pi_flat.shape[1]
    odim = w2.shape[1]
    kcols = k_gcn_cat.shape[1]
    full = lambda shape: pl.BlockSpec(shape, lambda i: (0, 0))
    oshape = jax.ShapeDtypeStruct((b, odim), jnp.float32)
    return pl.pallas_call(
        _tail_kernel,
        out_shape=(oshape, oshape, oshape),
        grid=(1,),
        in_specs=[
            full((b, 2 * _H2)), full((b, pi_in)),
            full((2 * _H2, kcols)),
            full((pi_in, kcols)),
            full((2 * _H * _H2, _H2)),
            full((_H2, _H)), full((_H2, _H)),
            full((1, 1)), full((1, 1)),
            full((_H, _H)), full((1, _H)), full((_H, odim)), full((1, odim)),
        ],
        out_specs=(full((b, odim)), full((b, odim)), full((b, odim))),
        compiler_params=pltpu.CompilerParams(
            dimension_semantics=("arbitrary",)),
    )(pooled, pi_flat, k_gcn_cat, k_pi_r, k_out,
      s_single, s_dual,
      jnp.reshape(b_single, (1, 1)).astype(jnp.float32),
      jnp.reshape(b_dual, (1, 1)).astype(jnp.float32),
      w1, b1.reshape(1, _H), w2, b2.reshape(1, odim))


def kernel(x0, x1, x2, x3, x4, x5, x6, x7,
           e0, e1, e2, e3, e4, e5, e6, e7,
           batch_PI,
           source_w, source_b,
           gcn0_w, gcn0_b, gcn1_w, gcn1_b,
           mlp0_0w, mlp0_0b, mlp0_1w, mlp0_1b,
           mlp1_0w, mlp1_0b, mlp1_1w, mlp1_1b,
           k_gcn_cat, cnn_w_mat, cnn_b, k_pi_cat, k_out,
           out_w1, out_b1, out_w2, out_b2,
           b_single, s_single, b_dual, s_dual):
    xs = (x0, x1, x2, x3, x4, x5, x6, x7)
    es = (e0, e1, e2, e3, e4, e5, e6, e7)
    b = len(xs)
    n = x0.shape[0]

    x_blocks = jnp.stack([x.astype(jnp.float32) for x in xs])

    words = []
    for e in es:
        src = e[0].astype(jnp.int32)
        dst = e[1].astype(jnp.int32)
        base = (src >> 7) * n + ((dst >> 3) << 3)
        words.append((base << 10) | ((dst & 7) << 7) | (src & 127))
    w_blocks = jnp.stack(words).reshape(b, -1, 128)

    pooled = _gcn_stack(
        w_blocks, x_blocks, source_w, source_b,
        gcn0_w, gcn0_b, mlp0_0w, mlp0_0b, mlp0_1w, mlp0_1b,
        gcn1_w, gcn1_b, mlp1_0w, mlp1_0b, mlp1_1w, mlp1_1b,
    ).reshape(b, 2 * _H2)

    bpi, c, pp, _ = batch_PI.shape
    oh = (pp - _CNN_K) // _CNN_S + 1
    cols_l = []
    for ky in range(_CNN_K):
        for kx in range(_CNN_K):
            cols_l.append(batch_PI[:, :, ky:ky + _CNN_S * (oh - 1) + 1:_CNN_S,
                                         kx:kx + _CNN_S * (oh - 1) + 1:_CNN_S])
    col = jnp.stack(cols_l, axis=2)
    col = col.transpose(0, 3, 4, 1, 2).reshape(bpi * oh * oh,
                                               c * _CNN_K * _CNN_K)
    emb = _conv_gemm(col.astype(jnp.float32), cnn_w_mat, cnn_b)
    pi_flat = emb.reshape(bpi, oh * oh * _H)
    k_pi_r = k_pi_cat.reshape(_H, oh * oh, -1).transpose(1, 0, 2) \
                     .reshape(oh * oh * _H, -1)

    score, score_gcn, score_top = _tail(
        pooled, pi_flat, k_gcn_cat, k_pi_r, k_out,
        s_single, s_dual, b_single, b_dual,
        out_w1, out_b1, out_w2, out_b2)
    return score, score_gcn, score_top


# speedup vs baseline: 1.5788x; 1.5788x over previous
import functools

import jax
import jax.numpy as jnp
from jax.experimental import pallas as pl
from jax.experimental.pallas import tpu as pltpu

_H = 8
_H2 = 64
_CNN_K, _CNN_S = 3, 2

_CH = 1024
_U = 2


def _gcn_kernel(w_ref, x_ref, ws_ref, bs_ref,
                wg0_ref, bg0_ref, m00w_ref, m00b_ref, m01w_ref, m01b_ref,
                wg1_ref, bg1_ref, m10w_ref, m10b_ref, m11w_ref, m11b_ref,
                o_ref, ae_ref, ao_ref, az_ref, k0_ref, k1_ref, sem_ref, *, n):
    f32 = jnp.float32
    segs = n // 128
    n_edges = w_ref.shape[1] * w_ref.shape[2]
    nch = n_edges // _CH

    ae_ref[...] = jnp.zeros((segs * n, 128), f32)
    ao_ref[...] = jnp.zeros((segs * n, 128), f32)
    az_ref[...] = jnp.zeros((segs * n, 128), f32)

    iota_sl = ((jax.lax.broadcasted_iota(jnp.int32, (8, 128), 0) << 7)
               | jax.lax.broadcasted_iota(jnp.int32, (8, 128), 1))

    def edge_copy(c, k_ref, s):
        return pltpu.make_async_copy(
            w_ref.at[0, pl.ds(c * 8, 8), :], k_ref, sem_ref.at[s])

    def do_chunk(k_ref):
        for g in range(0, _CH, _U):
            ab = (ae_ref, ao_ref, az_ref)[(g // _U) % 3]
            wv = [k_ref[(g + i) // 128, (g + i) % 128] for i in range(_U)]
            bases = [pl.multiple_of(w >> 10, 8) for w in wv]
            keys = [w & 1023 for w in wv]
            curs = [ab[pl.ds(bases[i], 8), :] for i in range(_U)]
            news = []
            for i in range(_U):
                val = curs[i]
                for j in range(i):
                    val = jnp.where(bases[j] == bases[i], news[j], val)
                news.append(jnp.where(iota_sl == keys[i], 1.0, val))
            for i in range(_U):
                ab[pl.ds(bases[i], 8), :] = news[i]

    edge_copy(0, k0_ref, 0).start()
    if nch > 1:
        edge_copy(1, k1_ref, 1).start()

    def ebody(c2, carry):
        c = c2 * 2
        edge_copy(c, k0_ref, 0).wait()
        do_chunk(k0_ref)

        @pl.when(c + 2 < nch)
        def _():
            edge_copy(c + 2, k0_ref, 0).start()

        edge_copy(c + 1, k1_ref, 1).wait()
        do_chunk(k1_ref)

        @pl.when(c + 3 < nch)
        def _():
            edge_copy(c + 3, k1_ref, 1).start()

        return carry

    jax.lax.fori_loop(0, nch // 2, ebody, 0)
    if nch % 2 == 1:
        edge_copy(nch - 1, k0_ref, 0).wait()
        do_chunk(k0_ref)

    ae_ref[...] = jnp.maximum(ae_ref[...],
                              jnp.maximum(ao_ref[...], az_ref[...]))

    x = x_ref[0]
    h0 = jnp.maximum(
        jnp.dot(x, ws_ref[...], preferred_element_type=f32) + bs_ref[...], 0.0)

    deg = None
    for s in range(segs):
        blk = ae_ref[s * n:(s + 1) * n, :]
        r = jnp.sum(blk, axis=1, keepdims=True)
        deg = r if deg is None else deg + r
    dinv = jax.lax.rsqrt(deg + 1.0)

    def aggmul(v):
        acc = None
        for s in range(segs):
            blk = ae_ref[s * n:(s + 1) * n, :]
            p = jnp.dot(blk, v[s * 128:(s + 1) * 128, :],
                        preferred_element_type=f32)
            acc = p if acc is None else acc + p
        return acc

    v0 = jnp.dot(h0, wg0_ref[...], preferred_element_type=f32) * dinv
    agg = (aggmul(v0) + v0) * dinv + bg0_ref[...]
    m = jnp.maximum(
        jnp.dot(agg, m00w_ref[...], preferred_element_type=f32) + m00b_ref[...], 0.0)
    h1 = jnp.dot(m, m01w_ref[...], preferred_element_type=f32) + m01b_ref[...]

    v1 = jnp.dot(h1, wg1_ref[...], preferred_element_type=f32) * dinv
    agg1 = (aggmul(v1) + v1) * dinv + bg1_ref[...]
    m1 = jnp.maximum(
        jnp.dot(agg1, m10w_ref[...], preferred_element_type=f32) + m10b_ref[...], 0.0)
    h2 = jnp.dot(m1, m11w_ref[...], preferred_element_type=f32) + m11b_ref[...]

    inv_n = 1.0 / n
    p1 = jnp.sum(h1, axis=0, keepdims=True) * inv_n
    p2 = jnp.sum(h2, axis=0, keepdims=True) * inv_n
    o_ref[0] = jnp.concatenate([p1, p2], axis=1)


def _gcn_stack(w_blocks, x_blocks, ws, bs, wg0, bg0, m00w, m00b, m01w, m01b,
               wg1, bg1, m10w, m10b, m11w, m11b):
    b, n, f = x_blocks.shape
    segs = n // 128
    ew = w_blocks.shape[1]
    wspec = lambda shape: pl.BlockSpec(shape, lambda i: (0, 0))
    return pl.pallas_call(
        functools.partial(_gcn_kernel, n=n),
        out_shape=jax.ShapeDtypeStruct((b, 1, 2 * _H2), jnp.float32),
        grid=(b,),
        in_specs=[
            pl.BlockSpec((1, ew, 128), lambda i: (i, 0, 0)),
            pl.BlockSpec((1, n, f), lambda i: (i, 0, 0)),
            wspec((f, _H)), wspec((1, _H)),
            wspec((_H, _H)), wspec((1, _H)),
            wspec((_H, _H)), wspec((1, _H)), wspec((_H, _H2)), wspec((1, _H2)),
            wspec((_H2, _H)), wspec((1, _H)),
            wspec((_H, _H)), wspec((1, _H)), wspec((_H, _H2)), wspec((1, _H2)),
        ],
        out_specs=pl.BlockSpec((1, 1, 2 * _H2), lambda i: (i, 0, 0)),
        scratch_shapes=[
            pltpu.VMEM((segs * n, 128), jnp.float32),
            pltpu.VMEM((segs * n, 128), jnp.float32),
            pltpu.VMEM((segs * n, 128), jnp.float32),
            pltpu.SMEM((8, 128), jnp.int32),
            pltpu.SMEM((8, 128), jnp.int32),
            pltpu.SemaphoreType.DMA((2,)),
        ],
        compiler_params=pltpu.CompilerParams(
            dimension_semantics=("parallel",),
            vmem_limit_bytes=60 * 1024 * 1024),
        cost_estimate=pl.CostEstimate(
            flops=2 * b * (2 * n * n * _H + n * (f + 4 * _H) * _H
                           + 2 * n * _H * _H2),
            transcendentals=0,
            bytes_accessed=4 * b * (ew * 128 + n * f + 2 * _H2)),
    )(w_blocks, x_blocks, ws, bs.reshape(1, _H),
      wg0, bg0.reshape(1, _H), m00w, m00b.reshape(1, _H),
      m01w, m01b.reshape(1, _H2),
      wg1, bg1.reshape(1, _H), m10w, m10b.reshape(1, _H),
      m11w, m11b.reshape(1, _H2))


def _conv_kernel(c_ref, w_ref, b_ref, o_ref):
    acc = jnp.dot(c_ref[...], w_ref[...], preferred_element_type=jnp.float32)
    o_ref[...] = jnp.maximum(acc + b_ref[...], 0.0)


def _conv_gemm(col, w, bias):
    r, k = col.shape
    return pl.pallas_call(
        _conv_kernel,
        out_shape=jax.ShapeDtypeStruct((r, _H), jnp.float32),
        grid=(1,),
        in_specs=[
            pl.BlockSpec((r, k), lambda i: (0, 0)),
            pl.BlockSpec((k, _H), lambda i: (0, 0)),
            pl.BlockSpec((1, _H), lambda i: (0, 0)),
        ],
        out_specs=pl.BlockSpec((r, _H), lambda i: (0, 0)),
        compiler_params=pltpu.CompilerParams(
            dimension_semantics=("arbitrary",)),
    )(col, w, bias.reshape(1, _H))


def _tail_kernel(pool_ref, pif_ref, kg_ref, kp_ref, ko_ref,
                 ss_ref, sd_ref, bs_ref, bd_ref,
                 w1_ref, b1_ref, w2_ref, b2_ref,
                 o_ref, og_ref, ot_ref):
    f32 = jnp.float32
    gcn_cat = jnp.dot(pool_ref[...], kg_ref[...], preferred_element_type=f32)
    pi_cat = jnp.dot(pif_ref[...], kp_ref[...], preferred_element_type=f32)
    z_gcn = gcn_cat[:, _H * _H2:_H * _H2 + _H2]
    z_pi = pi_cat[:, _H * _H2:_H * _H2 + _H2]
    z_dual = (
        jnp.dot(gcn_cat[:, :_H * _H2], ko_ref[:_H * _H2, :],
                preferred_element_type=f32)
        + jnp.dot(pi_cat[:, :_H * _H2], ko_ref[_H * _H2:, :],
                  preferred_element_type=f32))

    def head(z, b_att, s):
        att = jnp.maximum(z + b_att, 0.0)
        pooled = jnp.dot(att, s, preferred_element_type=f32)
        hdn = jnp.maximum(
            jnp.dot(pooled, w1_ref[...], preferred_element_type=f32)
            + b1_ref[...], 0.0)
        return jnp.dot(hdn, w2_ref[...], preferred_element_type=f32) + b2_ref[...]

    o_ref[...] = head(z_dual, bd_ref[...], sd_ref[...])
    og_ref[...] = head(z_gcn, bs_ref[...], ss_ref[...])
    ot_ref[...] = head(z_pi, bs_ref[...], ss_ref[...])


def _tail(pooled, pi_flat, k_gcn_cat, k_pi_r, k_out,
          s_single, s_dual, b_single, b_dual, w1, b1, w2, b2):
    b = pooled.shape[0]
    pi_in = pi_flat.shape[1]
    odim = w2.shape[1]
    kcols = k_gcn_cat.shape[1]
    full = lambda shape: pl.BlockSpec(shape, lambda i: (0, 0))
    oshape = jax.ShapeDtypeStruct((b, odim), jnp.float32)
    return pl.pallas_call(
        _tail_kernel,
        out_shape=(oshape, oshape, oshape),
        grid=(1,),
        in_specs=[
            full((b, 2 * _H2)), full((b, pi_in)),
            full((2 * _H2, kcols)),
            full((pi_in, kcols)),
            full((2 * _H * _H2, _H2)),
            full((_H2, _H)), full((_H2, _H)),
            full((1, 1)), full((1, 1)),
            full((_H, _H)), full((1, _H)), full((_H, odim)), full((1, odim)),
        ],
        out_specs=(full((b, odim)), full((b, odim)), full((b, odim))),
        compiler_params=pltpu.CompilerParams(
            dimension_semantics=("arbitrary",)),
    )(pooled, pi_flat, k_gcn_cat, k_pi_r, k_out,
      s_single, s_dual,
      jnp.reshape(b_single, (1, 1)).astype(jnp.float32),
      jnp.reshape(b_dual, (1, 1)).astype(jnp.float32),
      w1, b1.reshape(1, _H), w2, b2.reshape(1, odim))


def kernel(x0, x1, x2, x3, x4, x5, x6, x7,
           e0, e1, e2, e3, e4, e5, e6, e7,
           batch_PI,
           source_w, source_b,
           gcn0_w, gcn0_b, gcn1_w, gcn1_b,
           mlp0_0w, mlp0_0b, mlp0_1w, mlp0_1b,
           mlp1_0w, mlp1_0b, mlp1_1w, mlp1_1b,
           k_gcn_cat, cnn_w_mat, cnn_b, k_pi_cat, k_out,
           out_w1, out_b1, out_w2, out_b2,
           b_single, s_single, b_dual, s_dual):
    xs = (x0, x1, x2, x3, x4, x5, x6, x7)
    es = (e0, e1, e2, e3, e4, e5, e6, e7)
    b = len(xs)
    n = x0.shape[0]

    x_blocks = jnp.stack([x.astype(jnp.float32) for x in xs])

    words = []
    for e in es:
        src = e[0].astype(jnp.int32)
        dst = e[1].astype(jnp.int32)
        base = (src >> 7) * n + ((dst >> 3) << 3)
        words.append((base << 10) | ((dst & 7) << 7) | (src & 127))
    w_blocks = jnp.stack(words).reshape(b, -1, 128)

    pooled = _gcn_stack(
        w_blocks, x_blocks, source_w, source_b,
        gcn0_w, gcn0_b, mlp0_0w, mlp0_0b, mlp0_1w, mlp0_1b,
        gcn1_w, gcn1_b, mlp1_0w, mlp1_0b, mlp1_1w, mlp1_1b,
    ).reshape(b, 2 * _H2)

    bpi, c, pp, _ = batch_PI.shape
    oh = (pp - _CNN_K) // _CNN_S + 1
    cols_l = []
    for ky in range(_CNN_K):
        for kx in range(_CNN_K):
            cols_l.append(batch_PI[:, :, ky:ky + _CNN_S * (oh - 1) + 1:_CNN_S,
                                         kx:kx + _CNN_S * (oh - 1) + 1:_CNN_S])
    col = jnp.stack(cols_l, axis=2)
    col = col.transpose(0, 3, 4, 1, 2).reshape(bpi * oh * oh,
                                               c * _CNN_K * _CNN_K)
    emb = _conv_gemm(col.astype(jnp.float32), cnn_w_mat, cnn_b)
    pi_flat = emb.reshape(bpi, oh * oh * _H)
    k_pi_r = k_pi_cat.reshape(_H, oh * oh, -1).transpose(1, 0, 2) \
                     .reshape(oh * oh * _H, -1)

    score, score_gcn, score_top = _tail(
        pooled, pi_flat, k_gcn_cat, k_pi_r, k_out,
        s_single, s_dual, b_single, b_dual,
        out_w1, out_b1, out_w2, out_b2)
    return score, score_gcn, score_top


# speedup vs baseline: 1.5881x; 1.0059x over previous
import functools

import jax
import jax.numpy as jnp
from jax.experimental import pallas as pl
from jax.experimental.pallas import tpu as pltpu

_H = 8
_H2 = 64
_CNN_K, _CNN_S = 3, 2

_CH = 2048
_U = 2


def _gcn_kernel(w_ref, x_ref, ws_ref, bs_ref,
                wg0_ref, bg0_ref, m00w_ref, m00b_ref, m01w_ref, m01b_ref,
                wg1_ref, bg1_ref, m10w_ref, m10b_ref, m11w_ref, m11b_ref,
                o_ref, ae_ref, ao_ref, k0_ref, k1_ref, sem_ref, *, n):
    f32 = jnp.float32
    segs = n // 128
    n_edges = w_ref.shape[1] * w_ref.shape[2]
    nch = n_edges // _CH

    ae_ref[...] = jnp.zeros((segs * n, 128), f32)
    ao_ref[...] = jnp.zeros((segs * n, 128), f32)

    iota_sl = ((jax.lax.broadcasted_iota(jnp.int32, (8, 128), 0) << 7)
               | jax.lax.broadcasted_iota(jnp.int32, (8, 128), 1))

    def edge_copy(c, k_ref, s):
        return pltpu.make_async_copy(
            w_ref.at[0, pl.ds(c * (_CH // 128), _CH // 128), :], k_ref, sem_ref.at[s])

    def do_chunk(k_ref):
        for g in range(0, _CH, _U):
            ab = (ae_ref, ao_ref)[(g // _U) % 2]
            wv = [k_ref[(g + i) // 128, (g + i) % 128] for i in range(_U)]
            bases = [pl.multiple_of(w >> 10, 8) for w in wv]
            keys = [w & 1023 for w in wv]
            curs = [ab[pl.ds(bases[i], 8), :] for i in range(_U)]
            news = []
            for i in range(_U):
                val = curs[i]
                for j in range(i):
                    val = jnp.where(bases[j] == bases[i], news[j], val)
                news.append(jnp.where(iota_sl == keys[i], 1.0, val))
            for i in range(_U):
                ab[pl.ds(bases[i], 8), :] = news[i]

    edge_copy(0, k0_ref, 0).start()
    if nch > 1:
        edge_copy(1, k1_ref, 1).start()

    def ebody(c2, carry):
        c = c2 * 2
        edge_copy(c, k0_ref, 0).wait()
        do_chunk(k0_ref)

        @pl.when(c + 2 < nch)
        def _():
            edge_copy(c + 2, k0_ref, 0).start()

        edge_copy(c + 1, k1_ref, 1).wait()
        do_chunk(k1_ref)

        @pl.when(c + 3 < nch)
        def _():
            edge_copy(c + 3, k1_ref, 1).start()

        return carry

    jax.lax.fori_loop(0, nch // 2, ebody, 0)
    if nch % 2 == 1:
        edge_copy(nch - 1, k0_ref, 0).wait()
        do_chunk(k0_ref)

    ae_ref[...] = jnp.maximum(ae_ref[...], ao_ref[...])

    x = x_ref[0]
    h0 = jnp.maximum(
        jnp.dot(x, ws_ref[...], preferred_element_type=f32) + bs_ref[...], 0.0)

    deg = None
    for s in range(segs):
        blk = ae_ref[s * n:(s + 1) * n, :]
        r = jnp.sum(blk, axis=1, keepdims=True)
        deg = r if deg is None else deg + r
    dinv = jax.lax.rsqrt(deg + 1.0)

    def aggmul(v):
        acc = None
        for s in range(segs):
            blk = ae_ref[s * n:(s + 1) * n, :]
            p = jnp.dot(blk, v[s * 128:(s + 1) * 128, :],
                        preferred_element_type=f32)
            acc = p if acc is None else acc + p
        return acc

    v0 = jnp.dot(h0, wg0_ref[...], preferred_element_type=f32) * dinv
    agg = (aggmul(v0) + v0) * dinv + bg0_ref[...]
    m = jnp.maximum(
        jnp.dot(agg, m00w_ref[...], preferred_element_type=f32) + m00b_ref[...], 0.0)
    h1 = jnp.dot(m, m01w_ref[...], preferred_element_type=f32) + m01b_ref[...]

    v1 = jnp.dot(h1, wg1_ref[...], preferred_element_type=f32) * dinv
    agg1 = (aggmul(v1) + v1) * dinv + bg1_ref[...]
    m1 = jnp.maximum(
        jnp.dot(agg1, m10w_ref[...], preferred_element_type=f32) + m10b_ref[...], 0.0)
    h2 = jnp.dot(m1, m11w_ref[...], preferred_element_type=f32) + m11b_ref[...]

    inv_n = 1.0 / n
    p1 = jnp.sum(h1, axis=0, keepdims=True) * inv_n
    p2 = jnp.sum(h2, axis=0, keepdims=True) * inv_n
    o_ref[0] = jnp.concatenate([p1, p2], axis=1)


def _gcn_stack(w_blocks, x_blocks, ws, bs, wg0, bg0, m00w, m00b, m01w, m01b,
               wg1, bg1, m10w, m10b, m11w, m11b):
    b, n, f = x_blocks.shape
    segs = n // 128
    ew = w_blocks.shape[1]
    wspec = lambda shape: pl.BlockSpec(shape, lambda i: (0, 0))
    return pl.pallas_call(
        functools.partial(_gcn_kernel, n=n),
        out_shape=jax.ShapeDtypeStruct((b, 1, 2 * _H2), jnp.float32),
        grid=(b,),
        in_specs=[
            pl.BlockSpec((1, ew, 128), lambda i: (i, 0, 0)),
            pl.BlockSpec((1, n, f), lambda i: (i, 0, 0)),
            wspec((f, _H)), wspec((1, _H)),
            wspec((_H, _H)), wspec((1, _H)),
            wspec((_H, _H)), wspec((1, _H)), wspec((_H, _H2)), wspec((1, _H2)),
            wspec((_H2, _H)), wspec((1, _H)),
            wspec((_H, _H)), wspec((1, _H)), wspec((_H, _H2)), wspec((1, _H2)),
        ],
        out_specs=pl.BlockSpec((1, 1, 2 * _H2), lambda i: (i, 0, 0)),
        scratch_shapes=[
            pltpu.VMEM((segs * n, 128), jnp.float32),
            pltpu.VMEM((segs * n, 128), jnp.float32),
            pltpu.SMEM((_CH // 128, 128), jnp.int32),
            pltpu.SMEM((_CH // 128, 128), jnp.int32),
            pltpu.SemaphoreType.DMA((2,)),
        ],
        compiler_params=pltpu.CompilerParams(
            dimension_semantics=("parallel",),
            vmem_limit_bytes=60 * 1024 * 1024),
        cost_estimate=pl.CostEstimate(
            flops=2 * b * (2 * n * n * _H + n * (f + 4 * _H) * _H
                           + 2 * n * _H * _H2),
            transcendentals=0,
            bytes_accessed=4 * b * (ew * 128 + n * f + 2 * _H2)),
    )(w_blocks, x_blocks, ws, bs.reshape(1, _H),
      wg0, bg0.reshape(1, _H), m00w, m00b.reshape(1, _H),
      m01w, m01b.reshape(1, _H2),
      wg1, bg1.reshape(1, _H), m10w, m10b.reshape(1, _H),
      m11w, m11b.reshape(1, _H2))


def _conv_kernel(c_ref, w_ref, b_ref, o_ref):
    acc = jnp.dot(c_ref[...], w_ref[...], preferred_element_type=jnp.float32)
    o_ref[...] = jnp.maximum(acc + b_ref[...], 0.0)


def _conv_gemm(col, w, bias):
    r, k = col.shape
    return pl.pallas_call(
        _conv_kernel,
        out_shape=jax.ShapeDtypeStruct((r, _H), jnp.float32),
        grid=(1,),
        in_specs=[
            pl.BlockSpec((r, k), lambda i: (0, 0)),
            pl.BlockSpec((k, _H), lambda i: (0, 0)),
            pl.BlockSpec((1, _H), lambda i: (0, 0)),
        ],
        out_specs=pl.BlockSpec((r, _H), lambda i: (0, 0)),
        compiler_params=pltpu.CompilerParams(
            dimension_semantics=("arbitrary",)),
    )(col, w, bias.reshape(1, _H))


def _tail_kernel(pool_ref, pif_ref, kg_ref, kp_ref, ko_ref,
                 ss_ref, sd_ref, bs_ref, bd_ref,
                 w1_ref, b1_ref, w2_ref, b2_ref,
                 o_ref, og_ref, ot_ref):
    f32 = jnp.float32
    gcn_cat = jnp.dot(pool_ref[...], kg_ref[...], preferred_element_type=f32)
    pi_cat = jnp.dot(pif_ref[...], kp_ref[...], preferred_element_type=f32)
    z_gcn = gcn_cat[:, _H * _H2:_H * _H2 + _H2]
    z_pi = pi_cat[:, _H * _H2:_H * _H2 + _H2]
    z_dual = (
        jnp.dot(gcn_cat[:, :_H * _H2], ko_ref[:_H * _H2, :],
                preferred_element_type=f32)
        + jnp.dot(pi_cat[:, :_H * _H2], ko_ref[_H * _H2:, :],
                  preferred_element_type=f32))

    def head(z, b_att, s):
        att = jnp.maximum(z + b_att, 0.0)
        pooled = jnp.dot(att, s, preferred_element_type=f32)
        hdn = jnp.maximum(
            jnp.dot(pooled, w1_ref[...], preferred_element_type=f32)
            + b1_ref[...], 0.0)
        return jnp.dot(hdn, w2_ref[...], preferred_element_type=f32) + b2_ref[...]

    o_ref[...] = head(z_dual, bd_ref[...], sd_ref[...])
    og_ref[...] = head(z_gcn, bs_ref[...], ss_ref[...])
    ot_ref[...] = head(z_pi, bs_ref[...], ss_ref[...])


def _tail(pooled, pi_flat, k_gcn_cat, k_pi_r, k_out,
          s_single, s_dual, b_single, b_dual, w1, b1, w2, b2):
    b = pooled.shape[0]
    pi_in = pi_flat.shape[1]
    odim = w2.shape[1]
    kcols = k_gcn_cat.shape[1]
    full = lambda shape: pl.BlockSpec(shape, lambda i: (0, 0))
    oshape = jax.ShapeDtypeStruct((b, odim), jnp.float32)
    return pl.pallas_call(
        _tail_kernel,
        out_shape=(oshape, oshape, oshape),
        grid=(1,),
        in_specs=[
            full((b, 2 * _H2)), full((b, pi_in)),
            full((2 * _H2, kcols)),
            full((pi_in, kcols)),
            full((2 * _H * _H2, _H2)),
            full((_H2, _H)), full((_H2, _H)),
            full((1, 1)), full((1, 1)),
            full((_H, _H)), full((1, _H)), full((_H, odim)), full((1, odim)),
        ],
        out_specs=(full((b, odim)), full((b, odim)), full((b, odim))),
        compiler_params=pltpu.CompilerParams(
            dimension_semantics=("arbitrary",)),
    )(pooled, pi_flat, k_gcn_cat, k_pi_r, k_out,
      s_single, s_dual,
      jnp.reshape(b_single, (1, 1)).astype(jnp.float32),
      jnp.reshape(b_dual, (1, 1)).astype(jnp.float32),
      w1, b1.reshape(1, _H), w2, b2.reshape(1, odim))


def kernel(x0, x1, x2, x3, x4, x5, x6, x7,
           e0, e1, e2, e3, e4, e5, e6, e7,
           batch_PI,
           source_w, source_b,
           gcn0_w, gcn0_b, gcn1_w, gcn1_b,
           mlp0_0w, mlp0_0b, mlp0_1w, mlp0_1b,
           mlp1_0w, mlp1_0b, mlp1_1w, mlp1_1b,
           k_gcn_cat, cnn_w_mat, cnn_b, k_pi_cat, k_out,
           out_w1, out_b1, out_w2, out_b2,
           b_single, s_single, b_dual, s_dual):
    xs = (x0, x1, x2, x3, x4, x5, x6, x7)
    es = (e0, e1, e2, e3, e4, e5, e6, e7)
    b = len(xs)
    n = x0.shape[0]

    x_blocks = jnp.stack([x.astype(jnp.float32) for x in xs])

    words = []
    for e in es:
        src = e[0].astype(jnp.int32)
        dst = e[1].astype(jnp.int32)
        base = (src >> 7) * n + ((dst >> 3) << 3)
        words.append((base << 10) | ((dst & 7) << 7) | (src & 127))
    w_blocks = jnp.stack(words).reshape(b, -1, 128)

    pooled = _gcn_stack(
        w_blocks, x_blocks, source_w, source_b,
        gcn0_w, gcn0_b, mlp0_0w, mlp0_0b, mlp0_1w, mlp0_1b,
        gcn1_w, gcn1_b, mlp1_0w, mlp1_0b, mlp1_1w, mlp1_1b,
    ).reshape(b, 2 * _H2)

    bpi, c, pp, _ = batch_PI.shape
    oh = (pp - _CNN_K) // _CNN_S + 1
    cols_l = []
    for ky in range(_CNN_K):
        for kx in range(_CNN_K):
            cols_l.append(batch_PI[:, :, ky:ky + _CNN_S * (oh - 1) + 1:_CNN_S,
                                         kx:kx + _CNN_S * (oh - 1) + 1:_CNN_S])
    col = jnp.stack(cols_l, axis=2)
    col = col.transpose(0, 3, 4, 1, 2).reshape(bpi * oh * oh,
                                               c * _CNN_K * _CNN_K)
    emb = _conv_gemm(col.astype(jnp.float32), cnn_w_mat, cnn_b)
    pi_flat = emb.reshape(bpi, oh * oh * _H)
    k_pi_r = k_pi_cat.reshape(_H, oh * oh, -1).transpose(1, 0, 2) \
                     .reshape(oh * oh * _H, -1)

    score, score_gcn, score_top = _tail(
        pooled, pi_flat, k_gcn_cat, k_pi_r, k_out,
        s_single, s_dual, b_single, b_dual,
        out_w1, out_b1, out_w2, out_b2)
    return score, score_gcn, score_top
